# Initial kernel scaffold; baseline (speedup 1.0000x reference)
import jax
import jax.numpy as jnp
from jax.experimental import pallas as pl


def kernel(indices, table):
    raise NotImplementedError("write your pallas kernel here")


# speedup vs baseline: 1.0231x; 1.0231x over previous
import functools

import jax
import jax.numpy as jnp
from jax import lax
from jax.experimental import pallas as pl
from jax.experimental.pallas import tpu as pltpu
from jax.experimental.pallas import tpu_sc as plsc

_D = 32
_CHUNK = 128


@jax.jit
def _gather_sc(idx2d, table):
    n_rows, _ = idx2d.shape
    B = n_rows * _CHUNK
    info = plsc.get_sparse_core_info()
    nw = info.num_cores * info.num_subcores
    chunks_per_w = n_rows // nw

    mesh = plsc.VectorSubcoreMesh(core_axis_name="c", subcore_axis_name="s")

    @functools.partial(
        pl.kernel,
        mesh=mesh,
        out_type=jax.ShapeDtypeStruct((B, _D), jnp.float32),
        scratch_types=[
            pltpu.VMEM((chunks_per_w, _CHUNK), jnp.int32),
            pltpu.VMEM((_CHUNK, _D), jnp.float32),
            pltpu.SemaphoreType.DMA,
        ],
        compiler_params=pltpu.CompilerParams(use_tc_tiling_on_sc=False),
    )
    def k(idx_hbm, table_hbm, out_hbm, idx_v, rows_v, sem):
        wid = lax.axis_index("s") * info.num_cores + lax.axis_index("c")
        row0 = wid * chunks_per_w
        pltpu.sync_copy(idx_hbm.at[pl.ds(row0, chunks_per_w)], idx_v)

        def body(j, carry):
            pltpu.async_copy(table_hbm.at[idx_v.at[j]], rows_v, sem).wait()
            pltpu.sync_copy(
                rows_v, out_hbm.at[pl.ds((row0 + j) * _CHUNK, _CHUNK)]
            )
            return carry

        lax.fori_loop(0, chunks_per_w, body, 0)

    return k(idx2d, table)


def kernel(indices, table):
    B = indices.size
    idx2d = indices.reshape(B // _CHUNK, _CHUNK).astype(jnp.int32)
    out = _gather_sc(idx2d, table)
    return out.reshape(*indices.shape, _D)


# speedup vs baseline: 1.1112x; 1.0861x over previous
import functools

import jax
import jax.numpy as jnp
from jax import lax
from jax.experimental import pallas as pl
from jax.experimental.pallas import tpu as pltpu
from jax.experimental.pallas import tpu_sc as plsc

_D = 32
_CHUNK = 128
_NBUF = 20


@jax.jit
def _gather_sc(idx2d, table):
    n_rows, _ = idx2d.shape
    B = n_rows * _CHUNK
    info = plsc.get_sparse_core_info()
    nw = info.num_cores * info.num_subcores
    chunks_per_w = n_rows // nw
    n_outer = chunks_per_w // _NBUF

    mesh = plsc.VectorSubcoreMesh(core_axis_name="c", subcore_axis_name="s")

    @functools.partial(
        pl.kernel,
        mesh=mesh,
        out_type=jax.ShapeDtypeStruct((B, _D), jnp.float32),
        scratch_types=[
            pltpu.VMEM((chunks_per_w, _CHUNK), jnp.int32),
            pltpu.VMEM((_NBUF, _CHUNK, _D), jnp.float32),
            pltpu.SemaphoreType.DMA,
            pltpu.SemaphoreType.DMA,
        ],
        compiler_params=pltpu.CompilerParams(use_tc_tiling_on_sc=False),
    )
    def k(idx_hbm, table_hbm, out_hbm, idx_v, rows_v, gsem, wsem):
        wid = lax.axis_index("s") * info.num_cores + lax.axis_index("c")
        row0 = wid * chunks_per_w
        pltpu.sync_copy(idx_hbm.at[pl.ds(row0, chunks_per_w)], idx_v)

        def body(g, carry):
            j0 = g * _NBUF
            gd = [
                pltpu.async_copy(table_hbm.at[idx_v.at[j0 + b]],
                                 rows_v.at[b], gsem)
                for b in range(_NBUF)
            ]
            for d in gd:
                d.wait()
            wd = [
                pltpu.async_copy(
                    rows_v.at[b],
                    out_hbm.at[pl.ds((row0 + j0 + b) * _CHUNK, _CHUNK)],
                    wsem)
                for b in range(_NBUF)
            ]
            for d in wd:
                d.wait()
            return carry

        lax.fori_loop(0, n_outer, body, 0)

    return k(idx2d, table)


def kernel(indices, table):
    B = indices.size
    idx2d = indices.reshape(B // _CHUNK, _CHUNK).astype(jnp.int32)
    out = _gather_sc(idx2d, table)
    return out.reshape(*indices.shape, _D)


# speedup vs baseline: 1.7680x; 1.5911x over previous
import functools

import jax
import jax.numpy as jnp
from jax import lax
from jax.experimental import pallas as pl
from jax.experimental.pallas import tpu as pltpu
from jax.experimental.pallas import tpu_sc as plsc

_D = 32
_S = 50
_K = 16


@jax.jit
def _gather_sc(indices, table):
    n_outer_rows, _ = indices.shape
    info = plsc.get_sparse_core_info()
    nw = info.num_cores * info.num_subcores
    rows_per_w = n_outer_rows // nw
    n_rounds = rows_per_w // _K

    mesh = plsc.VectorSubcoreMesh(core_axis_name="c", subcore_axis_name="s")

    @functools.partial(
        pl.kernel,
        mesh=mesh,
        out_type=jax.ShapeDtypeStruct((n_outer_rows, _S, _D), jnp.float32),
        scratch_types=[
            pltpu.VMEM((rows_per_w, _S), jnp.int32),
            pltpu.VMEM((_K, _S, _D), jnp.float32),
            pltpu.SemaphoreType.DMA,
            pltpu.SemaphoreType.DMA,
        ],
        compiler_params=pltpu.CompilerParams(use_tc_tiling_on_sc=False),
    )
    def k(idx_hbm, table_hbm, out_hbm, idx_v, rows_v, gsem, wsem):
        wid = lax.axis_index("s") * info.num_cores + lax.axis_index("c")
        row0 = wid * rows_per_w
        pltpu.sync_copy(idx_hbm.at[pl.ds(row0, rows_per_w)], idx_v)

        def body(g, carry):
            r0 = g * _K
            gd = [
                pltpu.async_copy(table_hbm.at[idx_v.at[r0 + b]],
                                 rows_v.at[b], gsem)
                for b in range(_K)
            ]
            for d in gd:
                d.wait()
            wd = [
                pltpu.async_copy(rows_v.at[b], out_hbm.at[row0 + r0 + b],
                                 wsem)
                for b in range(_K)
            ]
            for d in wd:
                d.wait()
            return carry

        lax.fori_loop(0, n_rounds, body, 0)

    return k(indices, table)


def kernel(indices, table):
    return _gather_sc(indices.astype(jnp.int32), table)
